# baseline (speedup 1.0000x reference)
import functools

import jax
import jax.numpy as jnp
from jax import lax
from jax.experimental import pallas as pl
from jax.experimental.pallas import tpu as pltpu
from jax.experimental.pallas import tpu_sc as plsc

_V = 1000000
_D = 64
_NI = 16384
_NJ = 50
_NC, _NS = 2, 16
_NW = _NC * _NS
_IBLK = _NI // _NW
_Q = 128
_NT = _NJ * (_IBLK // _Q)


@functools.partial(
    pl.kernel,
    out_type=jax.ShapeDtypeStruct((_NJ, _D, _NI), jnp.float32),
    mesh=plsc.VectorSubcoreMesh(core_axis_name="c", subcore_axis_name="s"),
    scratch_types=[
        pltpu.VMEM((_NJ, _IBLK), jnp.int32),
        pltpu.VMEM((_Q,), jnp.int32),
        pltpu.VMEM((_Q,), jnp.int32),
        pltpu.VMEM((_Q,), jnp.int32),
        pltpu.VMEM((_Q,), jnp.int32),
        pltpu.VMEM((_Q, 2 * _D), jnp.float32),
        pltpu.VMEM((_Q, 2 * _D), jnp.float32),
        pltpu.VMEM((_D, _Q), jnp.float32),
        pltpu.VMEM((_D, _Q), jnp.float32),
        pltpu.SemaphoreType.DMA,
        pltpu.SemaphoreType.DMA,
        pltpu.SemaphoreType.DMA,
        pltpu.SemaphoreType.DMA,
    ],
    compiler_params=pltpu.CompilerParams(
        use_tc_tiling_on_sc=True, needs_layout_passes=False
    ),
)
def _emb_gather(idxT_hbm, pairs_hbm, outq_hbm, idxblk,
                sbuf0, sbuf1, hbuf0, hbuf1, gbuf0, gbuf1, qbuf0, qbuf1,
                gs0, gs1, ss0, ss1):
    wid = lax.axis_index("s") * _NC + lax.axis_index("c")
    i0 = wid * _IBLK
    sbuf = (sbuf0, sbuf1)
    hbuf = (hbuf0, hbuf1)
    gbuf = (gbuf0, gbuf1)
    qbuf = (qbuf0, qbuf1)
    gs = (gs0, gs1)
    ss = (ss0, ss1)
    lane = lax.iota(jnp.int32, 16)

    pltpu.sync_copy(idxT_hbm.at[:, pl.ds(i0, _IBLK)], idxblk)

    def prep_and_fire(t, b):
        j = t // (_IBLK // _Q)
        q = t % (_IBLK // _Q)

        def chunk(c, carry):
            v = idxblk[j, pl.ds(q * _Q + c * 16, 16)]
            sbuf[b][pl.ds(c * 16, 16)] = lax.shift_right_logical(v, 1)
            hbuf[b][pl.ds(c * 16, 16)] = lax.bitwise_and(v, 1)
            return carry

        lax.fori_loop(0, _Q // 16, chunk, 0)
        return pltpu.async_copy(pairs_hbm.at[sbuf[b]], gbuf[b], gs[b])

    def extract_and_store(t, b, ghandle):
        j = t // (_IBLK // _Q)
        q = t % (_IBLK // _Q)
        ghandle.wait()

        def drow(d, carry):
            def chunk(c, carry2):
                rows = lane + c * 16
                cols = hbuf[b][pl.ds(c * 16, 16)] * _D + d
                qbuf[b][d, pl.ds(c * 16, 16)] = plsc.load_gather(
                    gbuf[b], [rows, cols]
                )
                return carry2

            lax.fori_loop(0, _Q // 16, chunk, 0)
            return carry

        lax.fori_loop(0, _D, drow, 0)
        pltpu.async_copy(
            qbuf[b], outq_hbm.at[j, :, pl.ds(i0 + q * _Q, _Q)], ss[b]
        )

    def drain_store(b):
        pltpu.make_async_copy(
            qbuf[b], outq_hbm.at[0, :, pl.ds(i0, _Q)], ss[b]
        ).wait()

    def body(m, carry):
        h0 = prep_and_fire(2 * m, 0)
        h1 = prep_and_fire(2 * m + 1, 1)
        pl.when(m > 0)(lambda: drain_store(0))
        extract_and_store(2 * m, 0, h0)
        pl.when(m > 0)(lambda: drain_store(1))
        extract_and_store(2 * m + 1, 1, h1)
        return carry

    lax.fori_loop(0, _NT // 2, body, 0)
    drain_store(0)
    drain_store(1)


def kernel(input_, weight):
    idxT = input_.T.astype(jnp.int32)
    pairs = weight.reshape(_V // 2, 2 * _D)
    outq = _emb_gather(idxT, pairs)
    return outq.transpose(2, 0, 1)


# speedup vs baseline: 1.5017x; 1.5017x over previous
import functools

import jax
import jax.numpy as jnp
from jax import lax
from jax.experimental import pallas as pl
from jax.experimental.pallas import tpu as pltpu
from jax.experimental.pallas import tpu_sc as plsc

_V = 1000000
_D = 64
_NI = 16384
_NJ = 50
_NC, _NS = 2, 16
_NW = _NC * _NS
_IBLK = _NI // _NW
_Q = 128
_QPJ = _IBLK // _Q
_NT = _NJ * _QPJ


@functools.partial(
    pl.kernel,
    out_type=jax.ShapeDtypeStruct((_NJ, _D, _NI), jnp.float32),
    mesh=plsc.VectorSubcoreMesh(core_axis_name="c", subcore_axis_name="s"),
    scratch_types=[
        pltpu.VMEM((_NJ, _IBLK), jnp.int32),
        pltpu.VMEM((_Q,), jnp.int32),
        pltpu.VMEM((_Q,), jnp.int32),
        pltpu.VMEM((_Q,), jnp.int32),
        pltpu.VMEM((_Q,), jnp.int32),
        pltpu.VMEM((_Q, 2 * _D), jnp.float32),
        pltpu.VMEM((_Q, 2 * _D), jnp.float32),
        pltpu.VMEM((_D, _Q), jnp.float32),
        pltpu.VMEM((_D, _Q), jnp.float32),
        pltpu.SemaphoreType.DMA,
        pltpu.SemaphoreType.DMA,
        pltpu.SemaphoreType.DMA,
        pltpu.SemaphoreType.DMA,
    ],
    compiler_params=pltpu.CompilerParams(
        use_tc_tiling_on_sc=True, needs_layout_passes=False
    ),
)
def _emb_gather(idxT_hbm, pairs_hbm, outq_hbm, idxblk,
                sbuf0, sbuf1, hbuf0, hbuf1, gbuf0, gbuf1, qbuf0, qbuf1,
                gs0, gs1, ss0, ss1):
    wid = lax.axis_index("s") * _NC + lax.axis_index("c")
    i0 = wid * _IBLK
    sbuf = (sbuf0, sbuf1)
    hbuf = (hbuf0, hbuf1)
    gbuf = (gbuf0, gbuf1)
    qbuf = (qbuf0, qbuf1)
    gs = (gs0, gs1)
    ss = (ss0, ss1)
    lane = lax.iota(jnp.int32, 16)

    pltpu.sync_copy(idxT_hbm.at[:, pl.ds(i0, _IBLK)], idxblk)

    def prep_and_fire(t, b):
        j = t // _QPJ
        q = t % _QPJ
        base = q * _Q
        for c in range(_Q // 16):
            v = idxblk[j, pl.ds(base + c * 16, 16)]
            sbuf[b][pl.ds(c * 16, 16)] = lax.shift_right_logical(v, 1)
            hbuf[b][pl.ds(c * 16, 16)] = lax.shift_left(
                lax.bitwise_and(v, 1), 6
            )
        pltpu.async_copy(pairs_hbm.at[sbuf[b]], gbuf[b], gs[b])

    def wait_gather(b):
        pltpu.make_async_copy(pairs_hbm.at[sbuf[b]], gbuf[b], gs[b]).wait()

    def extract_and_store(t, b):
        j = t // _QPJ
        q = t % _QPJ

        def chunk_body(c, carry):
            rows = lane + c * 16
            hc = hbuf[b][pl.ds(c * 16, 16)]
            for d in range(_D):
                qbuf[b][d, pl.ds(c * 16, 16)] = plsc.load_gather(
                    gbuf[b], [rows, hc + d]
                )
            return carry

        lax.fori_loop(0, _Q // 16, chunk_body, 0)
        pltpu.async_copy(
            qbuf[b], outq_hbm.at[j, :, pl.ds(i0 + q * _Q, _Q)], ss[b]
        )

    def drain_store(b):
        pltpu.make_async_copy(
            qbuf[b], outq_hbm.at[0, :, pl.ds(i0, _Q)], ss[b]
        ).wait()

    prep_and_fire(0, 0)

    def body(p, carry):
        prep_and_fire(2 * p + 1, 1)
        wait_gather(0)
        pl.when(p > 0)(lambda: drain_store(0))
        extract_and_store(2 * p, 0)
        pl.when(p < _NT // 2 - 1)(lambda: prep_and_fire(2 * p + 2, 0))
        wait_gather(1)
        pl.when(p > 0)(lambda: drain_store(1))
        extract_and_store(2 * p + 1, 1)
        return carry

    lax.fori_loop(0, _NT // 2, body, 0)
    drain_store(0)
    drain_store(1)


def kernel(input_, weight):
    idxT = input_.T.astype(jnp.int32)
    pairs = weight.reshape(_V // 2, 2 * _D)
    outq = _emb_gather(idxT, pairs)
    return outq.transpose(2, 0, 1)


# speedup vs baseline: 2.0322x; 1.3533x over previous
import functools

import jax
import jax.numpy as jnp
from jax import lax
from jax.experimental import pallas as pl
from jax.experimental.pallas import tpu as pltpu
from jax.experimental.pallas import tpu_sc as plsc

_V = 1000000
_D = 64
_NI = 16384
_NJ = 50
_NC, _NS = 2, 16
_NW = _NC * _NS
_IBLK = _NI // _NW
_Q = 128
_QPJ = _IBLK // _Q
_NT = _NJ * _QPJ


@functools.partial(
    pl.kernel,
    out_type=jax.ShapeDtypeStruct((_NJ, _D, _NI), jnp.float32),
    mesh=plsc.VectorSubcoreMesh(core_axis_name="c", subcore_axis_name="s"),
    scratch_types=[
        pltpu.VMEM((_NJ, _IBLK), jnp.int32),
        pltpu.VMEM((_Q,), jnp.int32),
        pltpu.VMEM((_Q,), jnp.int32),
        pltpu.VMEM((_Q,), jnp.int32),
        pltpu.VMEM((_Q,), jnp.int32),
        pltpu.VMEM((_Q, 2 * _D), jnp.float32),
        pltpu.VMEM((_Q, 2 * _D), jnp.float32),
        pltpu.VMEM((_D, _Q), jnp.float32),
        pltpu.VMEM((_D, _Q), jnp.float32),
        pltpu.SemaphoreType.DMA,
        pltpu.SemaphoreType.DMA,
        pltpu.SemaphoreType.DMA,
        pltpu.SemaphoreType.DMA,
    ],
    compiler_params=pltpu.CompilerParams(
        use_tc_tiling_on_sc=True,
        needs_layout_passes=False,
        disable_bounds_checks=True,
    ),
)
def _emb_gather(idxT_hbm, pairs_hbm, outq_hbm, idxblk,
                sbuf0, sbuf1, hbuf0, hbuf1, gbuf0, gbuf1, qbuf0, qbuf1,
                gs0, gs1, ss0, ss1):
    wid = lax.axis_index("s") * _NC + lax.axis_index("c")
    i0 = wid * _IBLK
    sbuf = (sbuf0, sbuf1)
    hbuf = (hbuf0, hbuf1)
    gbuf = (gbuf0, gbuf1)
    qbuf = (qbuf0, qbuf1)
    gs = (gs0, gs1)
    ss = (ss0, ss1)
    lane = lax.iota(jnp.int32, 16)

    pltpu.sync_copy(idxT_hbm.at[:, pl.ds(i0, _IBLK)], idxblk)

    def prep_and_fire(t, b):
        j = t // _QPJ
        q = t % _QPJ
        base = q * _Q
        for c in range(_Q // 16):
            v = idxblk[j, pl.ds(base + c * 16, 16)]
            sbuf[b][pl.ds(c * 16, 16)] = lax.shift_right_logical(v, 1)
            hbuf[b][pl.ds(c * 16, 16)] = lax.shift_left(
                lax.bitwise_and(v, 1), 6
            )
        pltpu.async_copy(pairs_hbm.at[sbuf[b]], gbuf[b], gs[b])

    def wait_gather(b):
        pltpu.make_async_copy(pairs_hbm.at[sbuf[b]], gbuf[b], gs[b]).wait()

    def extract_and_store(t, b):
        j = t // _QPJ
        q = t % _QPJ

        @plsc.parallel_loop(0, _Q // 16, unroll=2)
        def chunk_body(c):
            rows = lane + c * 16
            hc = hbuf[b][pl.ds(c * 16, 16)]
            for d in range(_D):
                qbuf[b][d, pl.ds(c * 16, 16)] = plsc.load_gather(
                    gbuf[b], [rows, hc + d]
                )
        pltpu.async_copy(
            qbuf[b], outq_hbm.at[j, :, pl.ds(i0 + q * _Q, _Q)], ss[b]
        )

    def drain_store(b):
        pltpu.make_async_copy(
            qbuf[b], outq_hbm.at[0, :, pl.ds(i0, _Q)], ss[b]
        ).wait()

    prep_and_fire(0, 0)

    def body(p, carry):
        prep_and_fire(2 * p + 1, 1)
        wait_gather(0)
        pl.when(p > 0)(lambda: drain_store(0))
        extract_and_store(2 * p, 0)
        pl.when(p < _NT // 2 - 1)(lambda: prep_and_fire(2 * p + 2, 0))
        wait_gather(1)
        pl.when(p > 0)(lambda: drain_store(1))
        extract_and_store(2 * p + 1, 1)
        return carry

    lax.fori_loop(0, _NT // 2, body, 0)
    drain_store(0)
    drain_store(1)


def kernel(input_, weight):
    idxT = input_.T.astype(jnp.int32)
    pairs = weight.reshape(_V // 2, 2 * _D)
    outq = _emb_gather(idxT, pairs)
    return outq.transpose(2, 0, 1)
